# baseline (speedup 1.0000x reference)
import jax
import jax.numpy as jnp
from jax.experimental import pallas as pl
from jax.experimental.pallas import tpu as pltpu

ROWS, COLS = 4096, 8192
R = 128
SUB = 256
K = COLS // SUB


def _revcumsum_kernel(x_ref, o_ref):
    s = jax.lax.broadcasted_iota(jnp.int32, (SUB, SUB), 0)
    t = jax.lax.broadcasted_iota(jnp.int32, (SUB, SUB), 1)
    tri = (s >= t).astype(jnp.float32)

    carry = jnp.zeros((R, 1), jnp.float32)
    for k in range(K - 1, -1, -1):
        xs = x_ref[:, k * SUB:(k + 1) * SUB]
        p = jax.lax.dot(xs, tri, preferred_element_type=jnp.float32)
        o_ref[:, k * SUB:(k + 1) * SUB] = p + carry
        carry = carry + p[:, :1]


@jax.jit
def kernel(x):
    return pl.pallas_call(
        _revcumsum_kernel,
        grid=(ROWS // R,),
        in_specs=[pl.BlockSpec((R, COLS), lambda i: (i, 0))],
        out_specs=pl.BlockSpec((R, COLS), lambda i: (i, 0)),
        out_shape=jax.ShapeDtypeStruct((ROWS, COLS), jnp.float32),
        compiler_params=pltpu.CompilerParams(
            dimension_semantics=("parallel",)
        ),
    )(x)


# speedup vs baseline: 1.0333x; 1.0333x over previous
import jax
import jax.numpy as jnp
from jax.experimental import pallas as pl
from jax.experimental.pallas import tpu as pltpu

ROWS, COLS = 4096, 8192
R = 256
SUB = 256
K = COLS // SUB


def _revcumsum_kernel(x_ref, o_ref):
    s = jax.lax.broadcasted_iota(jnp.int32, (SUB, SUB), 0)
    t = jax.lax.broadcasted_iota(jnp.int32, (SUB, SUB), 1)
    tri = (s >= t).astype(jnp.float32)

    carry = jnp.zeros((R, 1), jnp.float32)
    for k in range(K - 1, -1, -1):
        xs = x_ref[:, k * SUB:(k + 1) * SUB]
        p = jax.lax.dot(xs, tri, preferred_element_type=jnp.float32)
        o_ref[:, k * SUB:(k + 1) * SUB] = p + carry
        carry = carry + p[:, :1]


@jax.jit
def kernel(x):
    return pl.pallas_call(
        _revcumsum_kernel,
        grid=(ROWS // R,),
        in_specs=[pl.BlockSpec((R, COLS), lambda i: (i, 0))],
        out_specs=pl.BlockSpec((R, COLS), lambda i: (i, 0)),
        out_shape=jax.ShapeDtypeStruct((ROWS, COLS), jnp.float32),
        compiler_params=pltpu.CompilerParams(
            dimension_semantics=("parallel",)
        ),
    )(x)
